# baseline (speedup 1.0000x reference)
import functools

import jax
import jax.numpy as jnp
from jax.experimental import pallas as pl

EMBED_DIM = 32
N_TOKENS = 32 * 1024
BLOCK = 1024
PADS = (1024, 128, 128)

_BIG_IDX = 2**30


def _quantize_one(x, rs, w, ws, pad_n):
    mm = jax.lax.dot_general(
        x, w, (((1,), (1,)), ((), ())),
        preferred_element_type=jnp.float32)
    d = (rs + ws) - 2.0 * mm
    m = jnp.min(d, axis=1, keepdims=True)
    lane = jax.lax.broadcasted_iota(jnp.int32, d.shape, 1)
    idx = jnp.min(jnp.where(d == m, lane, _BIG_IDX), axis=1, keepdims=True)
    oh = (lane == idx).astype(jnp.float32)
    return jax.lax.dot_general(
        oh, w, (((1,), (0,)), ((), ())),
        preferred_element_type=jnp.float32)


def _body(x_ref, rs_ref, w0_ref, ws0_ref, w1_ref, ws1_ref, w2_ref, ws2_ref,
          o0_ref, o1_ref, o2_ref):
    x = x_ref[...]
    rs = rs_ref[...]
    o0_ref[...] = _quantize_one(x, rs, w0_ref[...], ws0_ref[...], PADS[0])
    o1_ref[...] = _quantize_one(x, rs, w1_ref[...], ws1_ref[...], PADS[1])
    o2_ref[...] = _quantize_one(x, rs, w2_ref[...], ws2_ref[...], PADS[2])


@functools.partial(jax.jit, static_argnums=())
def kernel(embeds, W0, W1, W2):
    shape = embeds.shape
    flat = embeds.reshape(-1, EMBED_DIM)
    n = flat.shape[0]

    rs = jnp.sum(flat ** 2, axis=1, keepdims=True)

    ws = []
    wp = []
    for W, pad_n in zip((W0, W1, W2), PADS):
        c = W.shape[0]
        wsum = jnp.sum(W ** 2, axis=1)
        wsum = jnp.pad(wsum, (0, pad_n - c), constant_values=1e30)
        ws.append(wsum.reshape(1, pad_n))
        wp.append(jnp.pad(W, ((0, pad_n - c), (0, 0))))

    grid = (n // BLOCK,)
    rep = lambda i: (0, 0)
    out = pl.pallas_call(
        _body,
        grid=grid,
        in_specs=[
            pl.BlockSpec((BLOCK, EMBED_DIM), lambda i: (i, 0)),
            pl.BlockSpec((BLOCK, 1), lambda i: (i, 0)),
            pl.BlockSpec((PADS[0], EMBED_DIM), rep),
            pl.BlockSpec((1, PADS[0]), rep),
            pl.BlockSpec((PADS[1], EMBED_DIM), rep),
            pl.BlockSpec((1, PADS[1]), rep),
            pl.BlockSpec((PADS[2], EMBED_DIM), rep),
            pl.BlockSpec((1, PADS[2]), rep),
        ],
        out_specs=[
            pl.BlockSpec((BLOCK, EMBED_DIM), lambda i: (i, 0)),
            pl.BlockSpec((BLOCK, EMBED_DIM), lambda i: (i, 0)),
            pl.BlockSpec((BLOCK, EMBED_DIM), lambda i: (i, 0)),
        ],
        out_shape=[jax.ShapeDtypeStruct((n, EMBED_DIM), jnp.float32)] * 3,
    )(flat, rs, wp[0], ws[0], wp[1], ws[1], wp[2], ws[2])

    return tuple(o.reshape(shape) for o in out)
